# baseline (speedup 1.0000x reference)
import functools

import jax
import jax.numpy as jnp
import numpy as np
from jax import lax
from jax.experimental import pallas as pl
from jax.experimental.pallas import tpu as pltpu
from jax.experimental.pallas import tpu_sc as plsc

D = 128
K = 16
IMG_W = 1920.0
IMG_H = 1080.0
MAX_DISTANCE = 0.1
THRESH = MAX_DISTANCE * min(IMG_W, IMG_H)
BIG = 1e9

NUM_WORKERS = 32
CR = 8
BQ = 200


def _acos(x):
    ax = jnp.abs(x)
    p = -0.0012624911
    p = p * ax + 0.0066700901
    p = p * ax - 0.0170881256
    p = p * ax + 0.0308918810
    p = p * ax - 0.0501743046
    p = p * ax + 0.0889789874
    p = p * ax - 0.2145988016
    p = p * ax + 1.5707963050
    r = jnp.sqrt(1.0 - ax) * p
    return jnp.where(x < 0.0, np.pi - r, r)


def _prep_body(reid_ref, pos_ref, f_ref, xywh_ref, posn_ref):
    r = reid_ref[...]
    norm = jnp.sqrt(jnp.sum(r * r, axis=1, keepdims=True))
    f_ref[...] = r / (norm + 1e-12)
    p = pos_ref[...]
    cx = 0.5 * (p[:, 0:1] + p[:, 2:3])
    cy = 0.5 * (p[:, 1:2] + p[:, 3:4])
    w = p[:, 2:3] - p[:, 0:1]
    h = p[:, 3:4] - p[:, 1:2]
    xywh_ref[...] = jnp.concatenate([cx, cy, w, h], axis=1)
    posn_ref[...] = jnp.concatenate(
        [cx / IMG_W, cy / IMG_H, w / IMG_W, h / IMG_H], axis=1
    )


def _select_body(post_ref, pos_ref, topo_ref, idx_ref, vm_ref, *, row0, npad, bq):
    pid = pl.program_id(0)
    p = pos_ref[...]
    qx = 0.5 * (p[:, 0:1] + p[:, 2:3])
    qy = 0.5 * (p[:, 1:2] + p[:, 3:4])
    cx = 0.5 * (post_ref[0:1, :] + post_ref[2:3, :])
    cy = 0.5 * (post_ref[1:2, :] + post_ref[3:4, :])

    dx = qx - cx
    dy = qy - cy
    dist = jnp.sqrt(dx * dx + dy * dy + 1e-12)

    lane = lax.broadcasted_iota(jnp.int32, (bq, npad), 1)
    row_ids = row0 + pid * bq + lax.broadcasted_iota(jnp.int32, (bq, npad), 0)
    dist = jnp.where((dist <= THRESH) & (lane != row_ids), dist, BIG)

    nd_cols = []
    idx_cols = []
    for _ in range(K):
        minv = jnp.min(dist, axis=1, keepdims=True)
        cand = jnp.where(dist == minv, lane, npad)
        tie = jnp.min(cand, axis=1, keepdims=True)
        nd_cols.append(minv)
        idx_cols.append(tie)
        dist = jnp.where(lane == tie, 2.0 * BIG, dist)

    valid_cols = [nd < (BIG * 0.5) for nd in nd_cols]
    topo_ref[...] = jnp.concatenate(
        [jnp.where(v, nd, 0.0) / IMG_H for v, nd in zip(valid_cols, nd_cols)],
        axis=1,
    )
    idx_ref[...] = jnp.concatenate(
        [jnp.where(v, ix, 0) for v, ix in zip(valid_cols, idx_cols)], axis=1
    )
    vm_ref[...] = jnp.concatenate(
        [v.astype(jnp.float32) for v in valid_cols], axis=1
    )


def _sc_gather_body(
    cx_hbm, cy_hbm, idxf_hbm, f_hbm,
    nbx_hbm, nby_hbm, sim_hbm,
    cx_v, cy_v, idxf_v, fd_v, fga_v, fgb_v, nbx_v, nby_v, sim_v,
    sem_a, sem_b,
    *, rpw, row0,
):
    c = lax.axis_index("c")
    s = lax.axis_index("s")
    wid = s * 2 + c
    base = wid * rpw

    pltpu.sync_copy(cx_hbm, cx_v)
    pltpu.sync_copy(cy_hbm, cy_v)
    pltpu.sync_copy(idxf_hbm.at[pl.ds(base * K, rpw * K)], idxf_v)
    pltpu.sync_copy(f_hbm.at[pl.ds(row0 + base, rpw), :], fd_v)

    lanes = lax.broadcasted_iota(jnp.int32, (K,), 0)
    nchunks = rpw // CR

    def fire(ci, fg_v, sem):
        pltpu.async_copy(
            f_hbm.at[idxf_v.at[pl.ds(ci * CR * K, CR * K)]], fg_v, sem
        )

    def compute(ci, fg_v):
        r0 = ci * CR
        for rr in range(CR):
            idxr = idxf_v[pl.ds((r0 + rr) * K, K)]
            nbx_v[r0 + rr, :] = plsc.load_gather(cx_v, [idxr])
            nby_v[r0 + rr, :] = plsc.load_gather(cy_v, [idxr])
            simrow = jnp.zeros((K,), jnp.float32)
            for l in range(K):
                acc = fg_v[rr * K + l, pl.ds(0, 16)] * fd_v[r0 + rr, pl.ds(0, 16)]
                for ch in range(1, D // 16):
                    acc = acc + (
                        fg_v[rr * K + l, pl.ds(ch * 16, 16)]
                        * fd_v[r0 + rr, pl.ds(ch * 16, 16)]
                    )
                sval = jnp.sum(acc)
                simrow = jnp.where(lanes == l, sval, simrow)
            sim_v[r0 + rr, :] = simrow

    fire(0, fga_v, sem_a)

    def chunk_body(ci, carry):
        @pl.when(ci % 2 == 0)
        def _():
            @pl.when(ci + 1 < nchunks)
            def _():
                fire(ci + 1, fgb_v, sem_b)
            pltpu.make_async_copy(
                f_hbm.at[pl.ds(0, CR * K), :], fga_v, sem_a
            ).wait()
            compute(ci, fga_v)

        @pl.when(ci % 2 == 1)
        def _():
            @pl.when(ci + 1 < nchunks)
            def _():
                fire(ci + 1, fga_v, sem_a)
            pltpu.make_async_copy(
                f_hbm.at[pl.ds(0, CR * K), :], fgb_v, sem_b
            ).wait()
            compute(ci, fgb_v)

        return carry

    lax.fori_loop(0, nchunks, chunk_body, 0)

    pltpu.sync_copy(nbx_v, nbx_hbm.at[pl.ds(base, rpw), :])
    pltpu.sync_copy(nby_v, nby_hbm.at[pl.ds(base, rpw), :])
    pltpu.sync_copy(sim_v, sim_hbm.at[pl.ds(base, rpw), :])


def _epilogue_body(
    xywh_ref, nbx_ref, nby_ref, vm_ref, sim_in_ref,
    ang_ref, xd_ref, yd_ref, sim_ref,
):
    qx = xywh_ref[:, 0:1]
    qy = xywh_ref[:, 1:2]
    vm = vm_ref[...]
    vx = nbx_ref[...] - qx
    vy = nby_ref[...] - qy

    n1 = jnp.sqrt(vx * vx + vy * vy)
    ang_cols = []
    for k in range(K - 1):
        dot = vx[:, k:k + 1] * vx[:, k + 1:k + 2] + vy[:, k:k + 1] * vy[:, k + 1:k + 2]
        denom = n1[:, k:k + 1] * n1[:, k + 1:k + 2] + 1e-9
        cosang = jnp.clip(dot / denom, -1.0 + 1e-6, 1.0 - 1e-6)
        ang = _acos(cosang) * (180.0 / np.pi)
        pv = (vm[:, k:k + 1] * vm[:, k + 1:k + 2])
        ang_cols.append(ang * pv)
    ang_cols.append(jnp.zeros_like(ang_cols[0]))
    ang_ref[...] = jnp.concatenate(ang_cols, axis=1) / 360.0

    xd_ref[...] = (vx / IMG_W) * vm
    yd_ref[...] = (vy / IMG_H) * vm
    sim_ref[...] = sim_in_ref[...] * vm


def _run_select(post, positions, row0, rows, npad):
    bq = BQ if rows % BQ == 0 else rows
    grid = rows // bq
    body = functools.partial(_select_body, row0=row0, npad=npad, bq=bq)
    return pl.pallas_call(
        body,
        grid=(grid,),
        in_specs=[
            pl.BlockSpec((4, npad), lambda i: (0, 0)),
            pl.BlockSpec((bq, 4), lambda i: (i, 0)),
        ],
        out_specs=[
            pl.BlockSpec((bq, K), lambda i: (i, 0)),
            pl.BlockSpec((bq, K), lambda i: (i, 0)),
            pl.BlockSpec((bq, K), lambda i: (i, 0)),
        ],
        out_shape=[
            jax.ShapeDtypeStruct((rows, K), jnp.float32),
            jax.ShapeDtypeStruct((rows, K), jnp.int32),
            jax.ShapeDtypeStruct((rows, K), jnp.float32),
        ],
    )(post, lax.dynamic_slice_in_dim(positions, row0, rows, 0))


def _run_sc(cx_pad, cy_pad, f_pad, idx_safe, row0, rows):
    n_pad = ((rows + CR * NUM_WORKERS - 1) // (CR * NUM_WORKERS)) * (
        CR * NUM_WORKERS
    )
    rpw = n_pad // NUM_WORKERS
    npadr = cx_pad.shape[0]
    idx_flat = jnp.zeros((n_pad * K,), jnp.int32).at[: rows * K].set(
        idx_safe.reshape(-1)
    )
    body = functools.partial(_sc_gather_body, rpw=rpw, row0=row0)
    fn = pl.kernel(
        body,
        out_type=[
            jax.ShapeDtypeStruct((n_pad, K), jnp.float32),
            jax.ShapeDtypeStruct((n_pad, K), jnp.float32),
            jax.ShapeDtypeStruct((n_pad, K), jnp.float32),
        ],
        mesh=plsc.VectorSubcoreMesh(core_axis_name="c", subcore_axis_name="s"),
        compiler_params=pltpu.CompilerParams(needs_layout_passes=False),
        scratch_types=[
            pltpu.VMEM((npadr,), jnp.float32),
            pltpu.VMEM((npadr,), jnp.float32),
            pltpu.VMEM((rpw * K,), jnp.int32),
            pltpu.VMEM((rpw, D), jnp.float32),
            pltpu.VMEM((CR * K, D), jnp.float32),
            pltpu.VMEM((CR * K, D), jnp.float32),
            pltpu.VMEM((rpw, K), jnp.float32),
            pltpu.VMEM((rpw, K), jnp.float32),
            pltpu.VMEM((rpw, K), jnp.float32),
            pltpu.SemaphoreType.DMA,
            pltpu.SemaphoreType.DMA,
        ],
    )
    nbx_p, nby_p, sim_p = fn(cx_pad, cy_pad, idx_flat, f_pad)
    return nbx_p[:rows], nby_p[:rows], sim_p[:rows]


def _run_epilogue(pos_xywh, nbx, nby, vmask, sim_raw, row0, rows):
    bq = BQ if rows % BQ == 0 else rows
    grid = rows // bq
    return pl.pallas_call(
        _epilogue_body,
        grid=(grid,),
        in_specs=[pl.BlockSpec((bq, 4), lambda i: (i, 0))]
        + [pl.BlockSpec((bq, K), lambda i: (i, 0))] * 4,
        out_specs=[pl.BlockSpec((bq, K), lambda i: (i, 0))] * 4,
        out_shape=[jax.ShapeDtypeStruct((rows, K), jnp.float32)] * 4,
    )(
        lax.dynamic_slice_in_dim(pos_xywh, row0, rows, 0),
        nbx, nby, vmask, sim_raw,
    )


@jax.jit
def kernel(reid_features, positions, confs):
    n = reid_features.shape[0]
    npad = ((n + 127) // 128) * 128
    if n % (2 * BQ) == 0:
        halves = [(0, n // 2), (n // 2, n - n // 2)]
    elif n % BQ == 0:
        nb = n // BQ
        h0 = (nb // 2 + nb % 2) * BQ
        halves = [(0, h0), (h0, n - h0)]
    else:
        halves = [(0, n)]

    grid_all = n // BQ if n % BQ == 0 else 1
    bq_all = BQ if n % BQ == 0 else n

    f_norm, pos_xywh, pos_normed = pl.pallas_call(
        _prep_body,
        grid=(grid_all,),
        in_specs=[
            pl.BlockSpec((bq_all, D), lambda i: (i, 0)),
            pl.BlockSpec((bq_all, 4), lambda i: (i, 0)),
        ],
        out_specs=[
            pl.BlockSpec((bq_all, D), lambda i: (i, 0)),
            pl.BlockSpec((bq_all, 4), lambda i: (i, 0)),
            pl.BlockSpec((bq_all, 4), lambda i: (i, 0)),
        ],
        out_shape=[
            jax.ShapeDtypeStruct((n, D), jnp.float32),
            jax.ShapeDtypeStruct((n, 4), jnp.float32),
            jax.ShapeDtypeStruct((n, 4), jnp.float32),
        ],
    )(reid_features, positions)

    post = jnp.full((4, npad), 2e9, jnp.float32).at[:, :n].set(positions.T)
    cx_pad = jnp.zeros((npad,), jnp.float32).at[:n].set(pos_xywh[:, 0])
    cy_pad = jnp.zeros((npad,), jnp.float32).at[:n].set(pos_xywh[:, 1])
    f_pad = jnp.zeros((npad, D), jnp.float32).at[:n].set(f_norm)

    parts = []
    for row0, rows in halves:
        topo_h, idx_h, vm_h = _run_select(post, positions, row0, rows, npad)
        nbx_h, nby_h, sim_h = _run_sc(cx_pad, cy_pad, f_pad, idx_h, row0, rows)
        ang_h, xd_h, yd_h, simw_h = _run_epilogue(
            pos_xywh, nbx_h, nby_h, vm_h, sim_h, row0, rows
        )
        parts.append((topo_h, idx_h, ang_h, xd_h, yd_h, simw_h))

    topo_d, idx_safe, angles, xdiff, ydiff, simw = (
        jnp.concatenate([p[i] for p in parts], axis=0) for i in range(6)
    )

    node_feature = jnp.concatenate([f_norm, pos_normed, topo_d, angles], axis=1)

    src = idx_safe.reshape(-1)
    dst = jnp.repeat(jnp.arange(n, dtype=jnp.int32), K)
    edge_index = jnp.stack([src, dst], axis=0)

    edge_feature = jnp.stack(
        [xdiff.reshape(-1), ydiff.reshape(-1), simw.reshape(-1)], axis=1
    )
    return node_feature, edge_index, edge_feature


# speedup vs baseline: 1.0779x; 1.0779x over previous
import functools

import jax
import jax.numpy as jnp
import numpy as np
from jax import lax
from jax.experimental import pallas as pl
from jax.experimental.pallas import tpu as pltpu
from jax.experimental.pallas import tpu_sc as plsc

D = 128
K = 16
IMG_W = 1920.0
IMG_H = 1080.0
MAX_DISTANCE = 0.1
THRESH = MAX_DISTANCE * min(IMG_W, IMG_H)
BIG = 1e9

NUM_WORKERS = 32
CR = 8
BQ = 200


def _acos(x):
    ax = jnp.abs(x)
    p = -0.0012624911
    p = p * ax + 0.0066700901
    p = p * ax - 0.0170881256
    p = p * ax + 0.0308918810
    p = p * ax - 0.0501743046
    p = p * ax + 0.0889789874
    p = p * ax - 0.2145988016
    p = p * ax + 1.5707963050
    r = jnp.sqrt(1.0 - ax) * p
    return jnp.where(x < 0.0, np.pi - r, r)


def _prep_body(reid_ref, pos_ref, f_ref, xywh_ref, posn_ref):
    r = reid_ref[...]
    norm = jnp.sqrt(jnp.sum(r * r, axis=1, keepdims=True))
    f_ref[...] = r / (norm + 1e-12)
    p = pos_ref[...]
    cx = 0.5 * (p[:, 0:1] + p[:, 2:3])
    cy = 0.5 * (p[:, 1:2] + p[:, 3:4])
    w = p[:, 2:3] - p[:, 0:1]
    h = p[:, 3:4] - p[:, 1:2]
    xywh_ref[...] = jnp.concatenate([cx, cy, w, h], axis=1)
    posn_ref[...] = jnp.concatenate(
        [cx / IMG_W, cy / IMG_H, w / IMG_W, h / IMG_H], axis=1
    )


def _select_body(post_ref, pos_ref, topo_ref, idx_ref, vm_ref, *, row0, npad, bq):
    pid = pl.program_id(0)
    p = pos_ref[...]
    qx = 0.5 * (p[:, 0:1] + p[:, 2:3])
    qy = 0.5 * (p[:, 1:2] + p[:, 3:4])
    cx = 0.5 * (post_ref[0:1, :] + post_ref[2:3, :])
    cy = 0.5 * (post_ref[1:2, :] + post_ref[3:4, :])

    dx = qx - cx
    dy = qy - cy
    dist = jnp.sqrt(dx * dx + dy * dy + 1e-12)

    lane = lax.broadcasted_iota(jnp.int32, (bq, npad), 1)
    row_ids = row0 + pid * bq + lax.broadcasted_iota(jnp.int32, (bq, npad), 0)
    dist = jnp.where((dist <= THRESH) & (lane != row_ids), dist, BIG)

    nd_cols = []
    idx_cols = []
    for _ in range(K):
        minv = jnp.min(dist, axis=1, keepdims=True)
        cand = jnp.where(dist == minv, lane, npad)
        tie = jnp.min(cand, axis=1, keepdims=True)
        nd_cols.append(minv)
        idx_cols.append(tie)
        dist = jnp.where(lane == tie, 2.0 * BIG, dist)

    valid_cols = [nd < (BIG * 0.5) for nd in nd_cols]
    topo_ref[...] = jnp.concatenate(
        [jnp.where(v, nd, 0.0) / IMG_H for v, nd in zip(valid_cols, nd_cols)],
        axis=1,
    )
    idx_ref[...] = jnp.concatenate(
        [jnp.where(v, ix, 0) for v, ix in zip(valid_cols, idx_cols)], axis=1
    )
    vm_ref[...] = jnp.concatenate(
        [v.astype(jnp.float32) for v in valid_cols], axis=1
    )


def _sc_gather_body(
    cx_hbm, cy_hbm, idxf_hbm, f_hbm,
    nbx_hbm, nby_hbm, sim_hbm,
    cx_v, cy_v, idxf_v, fd_v, fga_v, fgb_v, nbx_v, nby_v, sim_v,
    sem_a, sem_b,
    *, rpw, row0,
):
    c = lax.axis_index("c")
    s = lax.axis_index("s")
    wid = s * 2 + c
    base = wid * rpw

    pltpu.sync_copy(cx_hbm, cx_v)
    pltpu.sync_copy(cy_hbm, cy_v)
    pltpu.sync_copy(idxf_hbm.at[pl.ds(base * K, rpw * K)], idxf_v)
    pltpu.sync_copy(f_hbm.at[pl.ds(row0 + base, rpw), :], fd_v)

    lanes = lax.broadcasted_iota(jnp.int32, (K,), 0)
    nchunks = rpw // CR

    def fire(ci, fg_v, sem):
        pltpu.async_copy(
            f_hbm.at[idxf_v.at[pl.ds(ci * CR * K, CR * K)]], fg_v, sem
        )

    def compute(ci, fg_v):
        r0 = ci * CR
        for rr in range(CR):
            idxr = idxf_v[pl.ds((r0 + rr) * K, K)]
            nbx_v[r0 + rr, :] = plsc.load_gather(cx_v, [idxr])
            nby_v[r0 + rr, :] = plsc.load_gather(cy_v, [idxr])
            simrow = jnp.zeros((K,), jnp.float32)
            for l in range(K):
                acc = fg_v[rr * K + l, pl.ds(0, 16)] * fd_v[r0 + rr, pl.ds(0, 16)]
                for ch in range(1, D // 16):
                    acc = acc + (
                        fg_v[rr * K + l, pl.ds(ch * 16, 16)]
                        * fd_v[r0 + rr, pl.ds(ch * 16, 16)]
                    )
                sval = jnp.sum(acc)
                simrow = jnp.where(lanes == l, sval, simrow)
            sim_v[r0 + rr, :] = simrow

    fire(0, fga_v, sem_a)

    def chunk_body(ci, carry):
        @pl.when(ci % 2 == 0)
        def _():
            @pl.when(ci + 1 < nchunks)
            def _():
                fire(ci + 1, fgb_v, sem_b)
            pltpu.make_async_copy(
                f_hbm.at[pl.ds(0, CR * K), :], fga_v, sem_a
            ).wait()
            compute(ci, fga_v)

        @pl.when(ci % 2 == 1)
        def _():
            @pl.when(ci + 1 < nchunks)
            def _():
                fire(ci + 1, fga_v, sem_a)
            pltpu.make_async_copy(
                f_hbm.at[pl.ds(0, CR * K), :], fgb_v, sem_b
            ).wait()
            compute(ci, fgb_v)

        return carry

    lax.fori_loop(0, nchunks, chunk_body, 0)

    pltpu.sync_copy(nbx_v, nbx_hbm.at[pl.ds(base, rpw), :])
    pltpu.sync_copy(nby_v, nby_hbm.at[pl.ds(base, rpw), :])
    pltpu.sync_copy(sim_v, sim_hbm.at[pl.ds(base, rpw), :])


def _epilogue_body(
    xywh_ref, nbx_ref, nby_ref, vm_ref, sim_in_ref,
    ang_ref, xd_ref, yd_ref, sim_ref,
):
    qx = xywh_ref[:, 0:1]
    qy = xywh_ref[:, 1:2]
    vm = vm_ref[...]
    vx = nbx_ref[...] - qx
    vy = nby_ref[...] - qy

    n1 = jnp.sqrt(vx * vx + vy * vy)
    ang_cols = []
    for k in range(K - 1):
        dot = vx[:, k:k + 1] * vx[:, k + 1:k + 2] + vy[:, k:k + 1] * vy[:, k + 1:k + 2]
        denom = n1[:, k:k + 1] * n1[:, k + 1:k + 2] + 1e-9
        cosang = jnp.clip(dot / denom, -1.0 + 1e-6, 1.0 - 1e-6)
        ang = _acos(cosang) * (180.0 / np.pi)
        pv = (vm[:, k:k + 1] * vm[:, k + 1:k + 2])
        ang_cols.append(ang * pv)
    ang_cols.append(jnp.zeros_like(ang_cols[0]))
    ang_ref[...] = jnp.concatenate(ang_cols, axis=1) / 360.0

    xd_ref[...] = (vx / IMG_W) * vm
    yd_ref[...] = (vy / IMG_H) * vm
    sim_ref[...] = sim_in_ref[...] * vm


def _run_select(post, positions, row0, rows, npad):
    bq = BQ if rows % BQ == 0 else rows
    grid = rows // bq
    body = functools.partial(_select_body, row0=row0, npad=npad, bq=bq)
    return pl.pallas_call(
        body,
        grid=(grid,),
        in_specs=[
            pl.BlockSpec((4, npad), lambda i: (0, 0)),
            pl.BlockSpec((bq, 4), lambda i: (i, 0)),
        ],
        out_specs=[
            pl.BlockSpec((bq, K), lambda i: (i, 0)),
            pl.BlockSpec((bq, K), lambda i: (i, 0)),
            pl.BlockSpec((bq, K), lambda i: (i, 0)),
        ],
        out_shape=[
            jax.ShapeDtypeStruct((rows, K), jnp.float32),
            jax.ShapeDtypeStruct((rows, K), jnp.int32),
            jax.ShapeDtypeStruct((rows, K), jnp.float32),
        ],
    )(post, lax.dynamic_slice_in_dim(positions, row0, rows, 0))


def _run_sc(cx_pad, cy_pad, f_pad, idx_safe, row0, rows):
    n_pad = ((rows + CR * NUM_WORKERS - 1) // (CR * NUM_WORKERS)) * (
        CR * NUM_WORKERS
    )
    rpw = n_pad // NUM_WORKERS
    npadr = cx_pad.shape[0]
    idx_flat = jnp.zeros((n_pad * K,), jnp.int32).at[: rows * K].set(
        idx_safe.reshape(-1)
    )
    body = functools.partial(_sc_gather_body, rpw=rpw, row0=row0)
    fn = pl.kernel(
        body,
        out_type=[
            jax.ShapeDtypeStruct((n_pad, K), jnp.float32),
            jax.ShapeDtypeStruct((n_pad, K), jnp.float32),
            jax.ShapeDtypeStruct((n_pad, K), jnp.float32),
        ],
        mesh=plsc.VectorSubcoreMesh(core_axis_name="c", subcore_axis_name="s"),
        compiler_params=pltpu.CompilerParams(needs_layout_passes=False),
        scratch_types=[
            pltpu.VMEM((npadr,), jnp.float32),
            pltpu.VMEM((npadr,), jnp.float32),
            pltpu.VMEM((rpw * K,), jnp.int32),
            pltpu.VMEM((rpw, D), jnp.float32),
            pltpu.VMEM((CR * K, D), jnp.float32),
            pltpu.VMEM((CR * K, D), jnp.float32),
            pltpu.VMEM((rpw, K), jnp.float32),
            pltpu.VMEM((rpw, K), jnp.float32),
            pltpu.VMEM((rpw, K), jnp.float32),
            pltpu.SemaphoreType.DMA,
            pltpu.SemaphoreType.DMA,
        ],
    )
    nbx_p, nby_p, sim_p = fn(cx_pad, cy_pad, idx_flat, f_pad)
    return nbx_p[:rows], nby_p[:rows], sim_p[:rows]


def _run_epilogue(pos_xywh, nbx, nby, vmask, sim_raw, row0, rows):
    bq = BQ if rows % BQ == 0 else rows
    grid = rows // bq
    return pl.pallas_call(
        _epilogue_body,
        grid=(grid,),
        in_specs=[pl.BlockSpec((bq, 4), lambda i: (i, 0))]
        + [pl.BlockSpec((bq, K), lambda i: (i, 0))] * 4,
        out_specs=[pl.BlockSpec((bq, K), lambda i: (i, 0))] * 4,
        out_shape=[jax.ShapeDtypeStruct((rows, K), jnp.float32)] * 4,
    )(
        lax.dynamic_slice_in_dim(pos_xywh, row0, rows, 0),
        nbx, nby, vmask, sim_raw,
    )


@jax.jit
def kernel(reid_features, positions, confs):
    n = reid_features.shape[0]
    npad = ((n + 127) // 128) * 128
    halves = [(0, n)]

    grid_all = n // BQ if n % BQ == 0 else 1
    bq_all = BQ if n % BQ == 0 else n

    f_norm, pos_xywh, pos_normed = pl.pallas_call(
        _prep_body,
        grid=(grid_all,),
        in_specs=[
            pl.BlockSpec((bq_all, D), lambda i: (i, 0)),
            pl.BlockSpec((bq_all, 4), lambda i: (i, 0)),
        ],
        out_specs=[
            pl.BlockSpec((bq_all, D), lambda i: (i, 0)),
            pl.BlockSpec((bq_all, 4), lambda i: (i, 0)),
            pl.BlockSpec((bq_all, 4), lambda i: (i, 0)),
        ],
        out_shape=[
            jax.ShapeDtypeStruct((n, D), jnp.float32),
            jax.ShapeDtypeStruct((n, 4), jnp.float32),
            jax.ShapeDtypeStruct((n, 4), jnp.float32),
        ],
    )(reid_features, positions)

    post = jnp.full((4, npad), 2e9, jnp.float32).at[:, :n].set(positions.T)
    cx_pad = jnp.zeros((npad,), jnp.float32).at[:n].set(pos_xywh[:, 0])
    cy_pad = jnp.zeros((npad,), jnp.float32).at[:n].set(pos_xywh[:, 1])
    f_pad = jnp.zeros((npad, D), jnp.float32).at[:n].set(f_norm)

    parts = []
    for row0, rows in halves:
        topo_h, idx_h, vm_h = _run_select(post, positions, row0, rows, npad)
        nbx_h, nby_h, sim_h = _run_sc(cx_pad, cy_pad, f_pad, idx_h, row0, rows)
        ang_h, xd_h, yd_h, simw_h = _run_epilogue(
            pos_xywh, nbx_h, nby_h, vm_h, sim_h, row0, rows
        )
        parts.append((topo_h, idx_h, ang_h, xd_h, yd_h, simw_h))

    topo_d, idx_safe, angles, xdiff, ydiff, simw = (
        jnp.concatenate([p[i] for p in parts], axis=0) for i in range(6)
    )

    node_feature = jnp.concatenate([f_norm, pos_normed, topo_d, angles], axis=1)

    src = idx_safe.reshape(-1)
    dst = jnp.repeat(jnp.arange(n, dtype=jnp.int32), K)
    edge_index = jnp.stack([src, dst], axis=0)

    edge_feature = jnp.stack(
        [xdiff.reshape(-1), ydiff.reshape(-1), simw.reshape(-1)], axis=1
    )
    return node_feature, edge_index, edge_feature
